# baseline (speedup 1.0000x reference)
import jax
import jax.numpy as jnp
from jax import lax
from jax.experimental import pallas as pl
from jax.experimental.pallas import tpu as pltpu
from jax.experimental.pallas import tpu_sc as plsc

NE = 8
TOPK = 2
D = 768
DFF = 2048
SEQ = 2048
EPS = 1e-6
BLK = 512
NBMAX = 16
NPAD = NBMAX * BLK

NC = 2
NS = 16
NW = NC * NS
TPW = SEQ // NW
CHUNK = 16


def _router_body(h_ref, lnw_ref, gw_ref,
                 hn_ref, dest0_ref, dest1_ref, wrep_ref, be_ref, nb_ref):
    h = h_ref[0]
    var = jnp.mean(h * h, axis=1, keepdims=True)
    hn = h * lax.rsqrt(var + EPS) * lnw_ref[...]
    hn_ref[...] = hn
    logits = lax.dot_general(
        hn.astype(jnp.bfloat16), gw_ref[...].astype(jnp.bfloat16),
        (((1,), (1,)), ((), ())), preferred_element_type=jnp.float32)

    iota8 = lax.broadcasted_iota(jnp.int32, (SEQ, NE), 1)
    m1 = jnp.max(logits, axis=1, keepdims=True)
    e0 = jnp.min(jnp.where(logits == m1, iota8, NE), axis=1, keepdims=True)
    l2 = jnp.where(iota8 == e0, -jnp.inf, logits)
    m2 = jnp.max(l2, axis=1, keepdims=True)
    e1 = jnp.min(jnp.where(l2 == m2, iota8, NE), axis=1, keepdims=True)
    dexp = jnp.exp(m2 - m1)
    w0 = 1.0 / (1.0 + dexp)
    w1 = dexp / (1.0 + dexp)
    wrep_ref[...] = jnp.concatenate(
        [jnp.broadcast_to(w0, (SEQ, 16)),
         jnp.broadcast_to(w1, (SEQ, 16)),
         jnp.zeros((SEQ, 96), jnp.float32)], axis=1)

    logits_t = lax.dot_general(
        gw_ref[...].astype(jnp.bfloat16), hn.astype(jnp.bfloat16),
        (((1,), (1,)), ((), ())), preferred_element_type=jnp.float32)
    iota8t = lax.broadcasted_iota(jnp.int32, (NE, SEQ), 0)
    m1t = jnp.max(logits_t, axis=0, keepdims=True)
    e0t = jnp.min(jnp.where(logits_t == m1t, iota8t, NE), axis=0,
                  keepdims=True)
    l2t = jnp.where(iota8t == e0t, -jnp.inf, logits_t)
    m2t = jnp.max(l2t, axis=0, keepdims=True)
    e1t = jnp.min(jnp.where(l2t == m2t, iota8t, NE), axis=0, keepdims=True)

    s = ((iota8t == e0t).astype(jnp.float32)
         + (iota8t == e1t).astype(jnp.float32))
    cum = s
    sh = 1
    while sh < SEQ:
        z = jnp.zeros((NE, sh), jnp.float32)
        cum = cum + jnp.concatenate([z, cum[:, :SEQ - sh]], axis=1)
        sh *= 2
    counts = cum[:, SEQ - 1:SEQ]
    cumx = cum - s

    ci = counts.astype(jnp.int32)
    pc = ((ci + (BLK - 1)) // BLK) * BLK
    ainc = pc
    for sh in (1, 2, 4):
        z = jnp.zeros((sh, 1), jnp.int32)
        ainc = ainc + jnp.concatenate([z, ainc[:NE - sh, :]], axis=0)
    aoff = (ainc - pc).astype(jnp.float32)

    rank0 = jnp.sum(jnp.where(iota8t == e0t, cumx, 0.0), axis=0, keepdims=True)
    rank1 = jnp.sum(jnp.where(iota8t == e1t, cumx, 0.0), axis=0, keepdims=True)
    off0 = jnp.sum(jnp.where(iota8t == e0t, aoff, 0.0), axis=0, keepdims=True)
    off1 = jnp.sum(jnp.where(iota8t == e1t, aoff, 0.0), axis=0, keepdims=True)
    dest0_ref[...] = jnp.broadcast_to(
        (off0 + rank0).astype(jnp.int32), (8, SEQ))
    dest1_ref[...] = jnp.broadcast_to(
        (off1 + rank1).astype(jnp.int32), (8, SEQ))

    bstart = lax.broadcasted_iota(jnp.int32, (1, NBMAX), 1) * BLK
    le = (jnp.broadcast_to(ainc, (NE, NBMAX))
          <= jnp.broadcast_to(bstart, (NE, NBMAX))).astype(jnp.int32)
    be = jnp.minimum(jnp.sum(le, axis=0, keepdims=True), NE - 1)
    be_ref[...] = be
    nb_ref[...] = ainc[NE - 1:NE, :] // BLK


def _run_router(hidden_2d, ln_w, gate_w, interpret=False):
    return pl.pallas_call(
        _router_body,
        out_shape=[
            jax.ShapeDtypeStruct((SEQ, D), jnp.float32),
            jax.ShapeDtypeStruct((8, SEQ), jnp.int32),
            jax.ShapeDtypeStruct((8, SEQ), jnp.int32),
            jax.ShapeDtypeStruct((SEQ, 128), jnp.float32),
            jax.ShapeDtypeStruct((1, NBMAX), jnp.int32),
            jax.ShapeDtypeStruct((1, 1), jnp.int32),
        ],
        interpret=interpret,
    )(hidden_2d, ln_w, gate_w)


def _scatter_body(hn_hbm, dest0_hbm, dest1_hbm, xs_hbm,
                  idx0_v, idx1_v, rows_v, sem):
    wid = lax.axis_index("s") * NC + lax.axis_index("c")
    base = wid * TPW
    pltpu.sync_copy(hn_hbm.at[pl.ds(base, TPW), :], rows_v)
    pltpu.sync_copy(dest0_hbm.at[0, pl.ds(base, TPW)], idx0_v)
    pltpu.sync_copy(dest1_hbm.at[0, pl.ds(base, TPW)], idx1_v)
    c0 = pltpu.async_copy(rows_v, xs_hbm.at[idx0_v], sem)
    c1 = pltpu.async_copy(rows_v, xs_hbm.at[idx1_v], sem)
    c0.wait()
    c1.wait()


def _run_scatter(hn, dest0, dest1):
    mesh = plsc.VectorSubcoreMesh(core_axis_name="c", subcore_axis_name="s")
    f = pl.kernel(
        _scatter_body,
        out_type=jax.ShapeDtypeStruct((NPAD, D), jnp.float32),
        mesh=mesh,
        scratch_types=[
            pltpu.VMEM((TPW,), jnp.int32),
            pltpu.VMEM((TPW,), jnp.int32),
            pltpu.VMEM((TPW, D), jnp.float32),
            pltpu.SemaphoreType.DMA,
        ],
    )
    return f(hn, dest0, dest1)


def _ffn_body(be_ref, nb_ref, x_ref, gw_ref, uw_ref, dw_ref, o_ref):
    b = pl.program_id(0)

    @pl.when(b < nb_ref[0, 0])
    def _():
        x = x_ref[...]
        acc = jnp.zeros((BLK, D), jnp.float32)
        half = DFF // 2
        for fi in range(2):
            gw = gw_ref[0, fi * half:(fi + 1) * half, :]
            uw = uw_ref[0, fi * half:(fi + 1) * half, :]
            g = lax.dot_general(x, gw, (((1,), (1,)), ((), ())),
                                preferred_element_type=jnp.float32)
            u = lax.dot_general(x, uw, (((1,), (1,)), ((), ())),
                                preferred_element_type=jnp.float32)
            hmid = g * jax.nn.sigmoid(g) * u
            dw = dw_ref[0, :, fi * half:(fi + 1) * half]
            acc = acc + lax.dot_general(hmid, dw, (((1,), (1,)), ((), ())),
                                        preferred_element_type=jnp.float32)
        o_ref[...] = acc


def _run_ffn(be, nb, xs, gate_up, down, interpret=False):
    grid_spec = pltpu.PrefetchScalarGridSpec(
        num_scalar_prefetch=2,
        grid=(NBMAX,),
        in_specs=[
            pl.BlockSpec((BLK, D),
                         lambda b, be, nb: (jnp.minimum(b, nb[0, 0] - 1), 0)),
            pl.BlockSpec((1, DFF, D), lambda b, be, nb:
                         (be[0, jnp.minimum(b, nb[0, 0] - 1)], 0, 0)),
            pl.BlockSpec((1, DFF, D), lambda b, be, nb:
                         (be[0, jnp.minimum(b, nb[0, 0] - 1)], 1, 0)),
            pl.BlockSpec((1, D, DFF), lambda b, be, nb:
                         (be[0, jnp.minimum(b, nb[0, 0] - 1)], 0, 0)),
        ],
        out_specs=pl.BlockSpec((BLK, D), lambda b, be, nb: (b, 0)),
    )
    return pl.pallas_call(
        _ffn_body,
        grid_spec=grid_spec,
        out_shape=jax.ShapeDtypeStruct((NPAD, D), jnp.float32),
        compiler_params=pltpu.CompilerParams(
            dimension_semantics=("arbitrary",)),
        interpret=interpret,
    )(be, nb, xs, gate_up, gate_up, down)


NCH = TPW // CHUNK


def _combine_body(part_hbm, dest0_hbm, dest1_hbm, wrep_hbm, res_hbm, out_hbm,
                  wrep_v, idx0_v, idx1_v,
                  p0a, p1a, resa, outa, p0b, p1b, resb, outb,
                  gsa, gsb, osa, osb):
    wid = lax.axis_index("s") * NC + lax.axis_index("c")
    base = wid * TPW
    pltpu.sync_copy(wrep_hbm.at[pl.ds(base, TPW), :], wrep_v)
    pltpu.sync_copy(dest0_hbm.at[0, pl.ds(base, TPW)], idx0_v)
    pltpu.sync_copy(dest1_hbm.at[0, pl.ds(base, TPW)], idx1_v)
    bufs = [(p0a, p1a, resa, outa, gsa, osa), (p0b, p1b, resb, outb, gsb, osb)]

    def fire(c):
        p0, p1, res, _, gs, _ = bufs[c % 2]
        sl = pl.ds(c * CHUNK, CHUNK)
        return (
            pltpu.async_copy(part_hbm.at[idx0_v.at[sl]], p0, gs),
            pltpu.async_copy(part_hbm.at[idx1_v.at[sl]], p1, gs),
            pltpu.async_copy(res_hbm.at[0, pl.ds(base + c * CHUNK, CHUNK), :],
                             res, gs),
        )

    descs = fire(0)
    wdescs = [None, None]
    for c in range(NCH):
        nxt = fire(c + 1) if c + 1 < NCH else None
        for dsc in descs:
            dsc.wait()
        if wdescs[c % 2] is not None:
            wdescs[c % 2].wait()
        p0, p1, res, out, _, osem = bufs[c % 2]

        def row(i, carry, c=c, p0=p0, p1=p1, res=res, out=out):
            r = c * CHUNK + i
            wv0 = wrep_v[r, pl.ds(0, 16)]
            wv1 = wrep_v[r, pl.ds(16, 16)]
            for j in range(D // 16):
                sl = pl.ds(j * 16, 16)
                out[i, sl] = (res[i, sl] + wv0 * p0[i, sl]
                              + wv1 * p1[i, sl])
            return carry

        lax.fori_loop(0, CHUNK, row, 0)
        wdescs[c % 2] = pltpu.async_copy(
            out, out_hbm.at[pl.ds(base + c * CHUNK, CHUNK), :], osem)
        descs = nxt
    wdescs[0].wait()
    wdescs[1].wait()


def _run_combine(part, dest0, dest1, wrep, residual):
    mesh = plsc.VectorSubcoreMesh(core_axis_name="c", subcore_axis_name="s")
    cbuf = pltpu.VMEM((CHUNK, D), jnp.float32)
    f = pl.kernel(
        _combine_body,
        out_type=jax.ShapeDtypeStruct((SEQ, D), jnp.float32),
        mesh=mesh,
        scratch_types=[
            pltpu.VMEM((TPW, 128), jnp.float32),
            pltpu.VMEM((TPW,), jnp.int32),
            pltpu.VMEM((TPW,), jnp.int32),
            cbuf, cbuf, cbuf, cbuf, cbuf, cbuf, cbuf, cbuf,
            pltpu.SemaphoreType.DMA,
            pltpu.SemaphoreType.DMA,
            pltpu.SemaphoreType.DMA,
            pltpu.SemaphoreType.DMA,
        ],
    )
    return f(part, dest0, dest1, wrep, residual)


def kernel(hidden_states, ln_weight, gate_weight, gate_up_stack, down_stack):
    B, S, Dm = hidden_states.shape
    hn, dest0, dest1, wrep, be, nb = _run_router(
        hidden_states, ln_weight.reshape(1, Dm), gate_weight)
    xs = _run_scatter(hn, dest0, dest1)
    part = _run_ffn(be, nb, xs, gate_up_stack, down_stack)
    out = _run_combine(part, dest0, dest1, wrep, hidden_states)
    return out.reshape(B, S, Dm)


# speedup vs baseline: 1.0037x; 1.0037x over previous
import jax
import jax.numpy as jnp
from jax import lax
from jax.experimental import pallas as pl
from jax.experimental.pallas import tpu as pltpu
from jax.experimental.pallas import tpu_sc as plsc

NE = 8
TOPK = 2
D = 768
DFF = 2048
SEQ = 2048
EPS = 1e-6
BLK = 512
NBMAX = 16
NPAD = NBMAX * BLK

NC = 2
NS = 16
NW = NC * NS
TPW = SEQ // NW
CHUNK = 16


def _router_body(h_ref, lnw_ref, gw_ref,
                 hn_ref, dest0_ref, dest1_ref, wrep_ref, be_ref, nb_ref):
    h = h_ref[0]
    var = jnp.mean(h * h, axis=1, keepdims=True)
    hn = h * lax.rsqrt(var + EPS) * lnw_ref[...]
    hn_ref[...] = hn
    logits = lax.dot_general(
        hn.astype(jnp.bfloat16), gw_ref[...].astype(jnp.bfloat16),
        (((1,), (1,)), ((), ())), preferred_element_type=jnp.float32)

    iota8 = lax.broadcasted_iota(jnp.int32, (SEQ, NE), 1)
    m1 = jnp.max(logits, axis=1, keepdims=True)
    e0 = jnp.min(jnp.where(logits == m1, iota8, NE), axis=1, keepdims=True)
    l2 = jnp.where(iota8 == e0, -jnp.inf, logits)
    m2 = jnp.max(l2, axis=1, keepdims=True)
    e1 = jnp.min(jnp.where(l2 == m2, iota8, NE), axis=1, keepdims=True)
    dexp = jnp.exp(m2 - m1)
    w0 = 1.0 / (1.0 + dexp)
    w1 = dexp / (1.0 + dexp)
    wrep_ref[...] = jnp.concatenate(
        [jnp.broadcast_to(w0, (SEQ, 16)),
         jnp.broadcast_to(w1, (SEQ, 16)),
         jnp.zeros((SEQ, 96), jnp.float32)], axis=1)

    logits_t = lax.dot_general(
        gw_ref[...].astype(jnp.bfloat16), hn.astype(jnp.bfloat16),
        (((1,), (1,)), ((), ())), preferred_element_type=jnp.float32)
    iota8t = lax.broadcasted_iota(jnp.int32, (NE, SEQ), 0)
    m1t = jnp.max(logits_t, axis=0, keepdims=True)
    e0t = jnp.min(jnp.where(logits_t == m1t, iota8t, NE), axis=0,
                  keepdims=True)
    l2t = jnp.where(iota8t == e0t, -jnp.inf, logits_t)
    m2t = jnp.max(l2t, axis=0, keepdims=True)
    e1t = jnp.min(jnp.where(l2t == m2t, iota8t, NE), axis=0, keepdims=True)

    s = ((iota8t == e0t).astype(jnp.float32)
         + (iota8t == e1t).astype(jnp.float32))
    cum = s
    sh = 1
    while sh < SEQ:
        z = jnp.zeros((NE, sh), jnp.float32)
        cum = cum + jnp.concatenate([z, cum[:, :SEQ - sh]], axis=1)
        sh *= 2
    counts = cum[:, SEQ - 1:SEQ]
    cumx = cum - s

    ci = counts.astype(jnp.int32)
    pc = ((ci + (BLK - 1)) // BLK) * BLK
    ainc = pc
    for sh in (1, 2, 4):
        z = jnp.zeros((sh, 1), jnp.int32)
        ainc = ainc + jnp.concatenate([z, ainc[:NE - sh, :]], axis=0)
    aoff = (ainc - pc).astype(jnp.float32)

    rank0 = jnp.sum(jnp.where(iota8t == e0t, cumx, 0.0), axis=0, keepdims=True)
    rank1 = jnp.sum(jnp.where(iota8t == e1t, cumx, 0.0), axis=0, keepdims=True)
    off0 = jnp.sum(jnp.where(iota8t == e0t, aoff, 0.0), axis=0, keepdims=True)
    off1 = jnp.sum(jnp.where(iota8t == e1t, aoff, 0.0), axis=0, keepdims=True)
    dest0_ref[...] = jnp.broadcast_to(
        (off0 + rank0).astype(jnp.int32), (8, SEQ))
    dest1_ref[...] = jnp.broadcast_to(
        (off1 + rank1).astype(jnp.int32), (8, SEQ))

    bstart = lax.broadcasted_iota(jnp.int32, (1, NBMAX), 1) * BLK
    le = (jnp.broadcast_to(ainc, (NE, NBMAX))
          <= jnp.broadcast_to(bstart, (NE, NBMAX))).astype(jnp.int32)
    be = jnp.minimum(jnp.sum(le, axis=0, keepdims=True), NE - 1)
    be_ref[...] = be
    nb_ref[...] = ainc[NE - 1:NE, :] // BLK


def _run_router(hidden_2d, ln_w, gate_w, interpret=False):
    return pl.pallas_call(
        _router_body,
        out_shape=[
            jax.ShapeDtypeStruct((SEQ, D), jnp.float32),
            jax.ShapeDtypeStruct((8, SEQ), jnp.int32),
            jax.ShapeDtypeStruct((8, SEQ), jnp.int32),
            jax.ShapeDtypeStruct((SEQ, 128), jnp.float32),
            jax.ShapeDtypeStruct((1, NBMAX), jnp.int32),
            jax.ShapeDtypeStruct((1, 1), jnp.int32),
        ],
        interpret=interpret,
    )(hidden_2d, ln_w, gate_w)


def _scatter_body(hn_hbm, dest0_hbm, dest1_hbm, xs_hbm,
                  idx0_v, idx1_v, rows_v, sem):
    wid = lax.axis_index("s") * NC + lax.axis_index("c")
    base = wid * TPW
    pltpu.sync_copy(hn_hbm.at[pl.ds(base, TPW), :], rows_v)
    pltpu.sync_copy(dest0_hbm.at[0, pl.ds(base, TPW)], idx0_v)
    pltpu.sync_copy(dest1_hbm.at[0, pl.ds(base, TPW)], idx1_v)
    c0 = pltpu.async_copy(rows_v, xs_hbm.at[idx0_v], sem)
    c1 = pltpu.async_copy(rows_v, xs_hbm.at[idx1_v], sem)
    c0.wait()
    c1.wait()


def _run_scatter(hn, dest0, dest1):
    mesh = plsc.VectorSubcoreMesh(core_axis_name="c", subcore_axis_name="s")
    f = pl.kernel(
        _scatter_body,
        out_type=jax.ShapeDtypeStruct((NPAD, D), jnp.float32),
        mesh=mesh,
        scratch_types=[
            pltpu.VMEM((TPW,), jnp.int32),
            pltpu.VMEM((TPW,), jnp.int32),
            pltpu.VMEM((TPW, D), jnp.float32),
            pltpu.SemaphoreType.DMA,
        ],
    )
    return f(hn, dest0, dest1)


def _ffn_body(be_ref, nb_ref, x_ref, gw_ref, uw_ref, dw_ref, o_ref):
    b = pl.program_id(0)

    @pl.when(b < nb_ref[0, 0])
    def _():
        x = x_ref[...]
        acc = jnp.zeros((BLK, D), jnp.float32)
        half = DFF // 2
        for fi in range(2):
            gw = gw_ref[0, fi * half:(fi + 1) * half, :]
            uw = uw_ref[0, fi * half:(fi + 1) * half, :]
            g = lax.dot_general(x, gw, (((1,), (1,)), ((), ())),
                                preferred_element_type=jnp.float32)
            u = lax.dot_general(x, uw, (((1,), (1,)), ((), ())),
                                preferred_element_type=jnp.float32)
            hmid = g * jax.nn.sigmoid(g) * u
            dw = dw_ref[0, :, fi * half:(fi + 1) * half]
            acc = acc + lax.dot_general(hmid, dw, (((1,), (1,)), ((), ())),
                                        preferred_element_type=jnp.float32)
        o_ref[...] = acc


def _run_ffn(be, nb, xs, gate_up, down, interpret=False):
    grid_spec = pltpu.PrefetchScalarGridSpec(
        num_scalar_prefetch=2,
        grid=(NBMAX,),
        in_specs=[
            pl.BlockSpec((BLK, D),
                         lambda b, be, nb: (jnp.minimum(b, nb[0, 0] - 1), 0)),
            pl.BlockSpec((1, DFF, D), lambda b, be, nb:
                         (be[0, jnp.minimum(b, nb[0, 0] - 1)], 0, 0)),
            pl.BlockSpec((1, DFF, D), lambda b, be, nb:
                         (be[0, jnp.minimum(b, nb[0, 0] - 1)], 1, 0)),
            pl.BlockSpec((1, D, DFF), lambda b, be, nb:
                         (be[0, jnp.minimum(b, nb[0, 0] - 1)], 0, 0)),
        ],
        out_specs=pl.BlockSpec((BLK, D), lambda b, be, nb: (b, 0)),
    )
    return pl.pallas_call(
        _ffn_body,
        grid_spec=grid_spec,
        out_shape=jax.ShapeDtypeStruct((NPAD, D), jnp.float32),
        compiler_params=pltpu.CompilerParams(
            dimension_semantics=("arbitrary",)),
        interpret=interpret,
    )(be, nb, xs, gate_up, gate_up, down)


NCH = TPW // CHUNK


def _combine_body(part_hbm, dest0_hbm, dest1_hbm, wrep_hbm, res_hbm, out_hbm,
                  wrep_v, idx0_v, idx1_v,
                  p0a, p1a, resa, outa, p0b, p1b, resb, outb,
                  gsa, gsb, osa, osb):
    wid = lax.axis_index("s") * NC + lax.axis_index("c")
    base = wid * TPW
    pltpu.sync_copy(wrep_hbm.at[pl.ds(base, TPW), :], wrep_v)
    pltpu.sync_copy(dest0_hbm.at[0, pl.ds(base, TPW)], idx0_v)
    pltpu.sync_copy(dest1_hbm.at[0, pl.ds(base, TPW)], idx1_v)
    bufs = [(p0a, p1a, resa, outa, gsa, osa), (p0b, p1b, resb, outb, gsb, osb)]

    def fire(c):
        p0, p1, res, _, gs, _ = bufs[c % 2]
        sl = pl.ds(c * CHUNK, CHUNK)
        return (
            pltpu.async_copy(part_hbm.at[idx0_v.at[sl]], p0, gs),
            pltpu.async_copy(part_hbm.at[idx1_v.at[sl]], p1, gs),
            pltpu.async_copy(res_hbm.at[0, pl.ds(base + c * CHUNK, CHUNK), :],
                             res, gs),
        )

    descs = fire(0)
    wdescs = [None, None]
    for c in range(NCH):
        nxt = fire(c + 1) if c + 1 < NCH else None
        for dsc in descs:
            dsc.wait()
        if wdescs[c % 2] is not None:
            wdescs[c % 2].wait()
        p0, p1, res, out, _, osem = bufs[c % 2]

        @plsc.parallel_loop(0, CHUNK, unroll=2)
        def _row(i, c=c, p0=p0, p1=p1, res=res, out=out):
            r = c * CHUNK + i
            wv0 = wrep_v[r, pl.ds(0, 16)]
            wv1 = wrep_v[r, pl.ds(16, 16)]
            for j in range(D // 16):
                sl = pl.ds(j * 16, 16)
                out[i, sl] = (res[i, sl] + wv0 * p0[i, sl]
                              + wv1 * p1[i, sl])
        wdescs[c % 2] = pltpu.async_copy(
            out, out_hbm.at[pl.ds(base + c * CHUNK, CHUNK), :], osem)
        descs = nxt
    wdescs[0].wait()
    wdescs[1].wait()


def _run_combine(part, dest0, dest1, wrep, residual):
    mesh = plsc.VectorSubcoreMesh(core_axis_name="c", subcore_axis_name="s")
    cbuf = pltpu.VMEM((CHUNK, D), jnp.float32)
    f = pl.kernel(
        _combine_body,
        out_type=jax.ShapeDtypeStruct((SEQ, D), jnp.float32),
        mesh=mesh,
        scratch_types=[
            pltpu.VMEM((TPW, 128), jnp.float32),
            pltpu.VMEM((TPW,), jnp.int32),
            pltpu.VMEM((TPW,), jnp.int32),
            cbuf, cbuf, cbuf, cbuf, cbuf, cbuf, cbuf, cbuf,
            pltpu.SemaphoreType.DMA,
            pltpu.SemaphoreType.DMA,
            pltpu.SemaphoreType.DMA,
            pltpu.SemaphoreType.DMA,
        ],
    )
    return f(part, dest0, dest1, wrep, residual)


def kernel(hidden_states, ln_weight, gate_weight, gate_up_stack, down_stack):
    B, S, Dm = hidden_states.shape
    hn, dest0, dest1, wrep, be, nb = _run_router(
        hidden_states, ln_weight.reshape(1, Dm), gate_weight)
    xs = _run_scatter(hn, dest0, dest1)
    part = _run_ffn(be, nb, xs, gate_up_stack, down_stack)
    out = _run_combine(part, dest0, dest1, wrep, hidden_states)
    return out.reshape(B, S, Dm)
